# Initial kernel scaffold; baseline (speedup 1.0000x reference)
import jax
import jax.numpy as jnp
from jax.experimental import pallas as pl


def kernel(x, edge_attr, pos, W1, W2, W2_1, W3, W4, W5, W6, W7, g1, b1, g2, b2, g2_1, b2_1, g3, b3, g4, b4, g5, b5, g6, b6, g7, b7, Wfc, edge_index, batch):
    raise NotImplementedError("write your pallas kernel here")


# speedup vs baseline: 1.0000x; 1.0000x over previous
import jax
import jax.numpy as jnp
from jax.experimental import pallas as pl
from jax.experimental.pallas import tpu as pltpu

S0 = 16.0 / 346.0
S1 = 12.0 / 260.0
NGRID = 22
NG2 = NGRID * NGRID


def _spline_conv(x, src, dst, pseudo, W):
    u0 = pseudo[:, 0]
    u1 = pseudo[:, 1]
    b = jnp.stack([(1.0 - u0) * (1.0 - u1), u0 * (1.0 - u1), (1.0 - u0) * u1, u0 * u1], axis=1)
    xs = jnp.take(x, src, axis=0)
    msg = b[:, 0:1] * (xs @ W[0])
    for k in range(1, 4):
        msg = msg + b[:, k:k + 1] * (xs @ W[k])
    n = x.shape[0]
    agg = jax.ops.segment_sum(msg, dst, num_segments=n)
    deg = jax.ops.segment_sum(jnp.ones((msg.shape[0],), x.dtype), dst, num_segments=n)
    return agg / jnp.clip(deg, 1.0, None)[:, None]


def _spline_conv_masked(x, src, dst, pseudo, W, w):
    u0 = pseudo[:, 0]
    u1 = pseudo[:, 1]
    b = jnp.stack([(1.0 - u0) * (1.0 - u1), u0 * (1.0 - u1), (1.0 - u0) * u1, u0 * u1], axis=1)
    xs = jnp.take(x, src, axis=0)
    msg = b[:, 0:1] * (xs @ W[0])
    for k in range(1, 4):
        msg = msg + b[:, k:k + 1] * (xs @ W[k])
    n = x.shape[0]
    msg = msg * w[:, None]
    agg = jax.ops.segment_sum(msg, dst, num_segments=n)
    deg = jax.ops.segment_sum(w, dst, num_segments=n)
    return agg / jnp.clip(deg, 1.0, None)[:, None]


def _bn(x, g, b):
    m = jnp.mean(x, axis=0)
    v = jnp.var(x, axis=0)
    return g * (x - m) / jnp.sqrt(v + 1e-5) + b


def _bn_masked(x, g, b, occf, m_count):
    m = jnp.sum(x * occf[:, None], axis=0) / m_count
    v = jnp.sum(((x - m) ** 2) * occf[:, None], axis=0) / m_count
    return g * (x - m) / jnp.sqrt(v + 1e-5) + b


def _fc_kernel(xf_ref, w_ref, o_ref):
    o_ref[...] = xf_ref[...] @ w_ref[...]


def _fc(xf, Wfc):
    return pl.pallas_call(
        _fc_kernel,
        out_shape=jax.ShapeDtypeStruct((1, Wfc.shape[1]), jnp.float32),
    )(xf, Wfc)


def kernel(x, edge_attr, pos, W1, W2, W2_1, W3, W4, W5, W6, W7, g1, b1, g2, b2, g2_1, b2_1, g3, b3, g4, b4, g5, b5, g6, b6, g7, b7, Wfc, edge_index, batch):
    c = jnp.floor(pos[:, 0] / S0).astype(jnp.int32) + NGRID * jnp.floor(pos[:, 1] / S1).astype(jnp.int32)
    ce0 = jnp.take(c, edge_index[0])
    ce1 = jnp.take(c, edge_index[1])
    mask = (ce0 != ce1).astype(pos.dtype)
    keys = ce0 * NG2 + ce1
    pres = jax.ops.segment_sum(mask, keys, num_segments=NG2 * NG2) > 0
    inv = c
    src = edge_index[0]
    dst = edge_index[1]

    h = jax.nn.elu(_spline_conv(x, src, dst, edge_attr, W1)); h = _bn(h, g1, b1)
    h = jax.nn.elu(_spline_conv(h, src, dst, edge_attr, W2)); h = _bn(h, g2, b2)
    h = jax.nn.elu(_spline_conv(h, src, dst, edge_attr, W2_1)); h = _bn(h, g2_1, b2_1)
    hsc = h
    h = jax.nn.elu(_spline_conv(h, src, dst, edge_attr, W3)); h = _bn(h, g3, b3)
    h = jax.nn.elu(_spline_conv(h, src, dst, edge_attr, W4)); h = _bn(h, g4, b4)
    h = h + hsc
    h = jax.nn.elu(_spline_conv(h, src, dst, edge_attr, W5)); h = _bn(h, g5, b5)

    cnt = jax.ops.segment_sum(jnp.ones((h.shape[0],), h.dtype), inv, num_segments=NG2)
    occ = cnt > 0
    occf = occ.astype(h.dtype)
    m_count = jnp.sum(occf)
    h2 = jax.ops.segment_max(h, inv, num_segments=NG2)
    h2 = jnp.where(occ[:, None], h2, 0.0)
    pos2 = jnp.where(occ[:, None], jax.ops.segment_sum(pos, inv, num_segments=NG2) / cnt[:, None], 0.0)
    src2 = jnp.arange(NG2 * NG2, dtype=jnp.int32) // NG2
    dst2 = jnp.arange(NG2 * NG2, dtype=jnp.int32) % NG2
    presf = pres.astype(h.dtype)
    d = jnp.take(pos2, src2, axis=0) - jnp.take(pos2, dst2, axis=0)
    ea2 = d / (2.0 * jnp.max(jnp.abs(d) * presf[:, None])) + 0.5
    hsc = h2
    h2 = jax.nn.elu(_spline_conv_masked(h2, src2, dst2, ea2, W6, presf)); h2 = _bn_masked(h2, g6, b6, occf, m_count)
    h2 = jax.nn.elu(_spline_conv_masked(h2, src2, dst2, ea2, W7, presf)); h2 = _bn_masked(h2, g7, b7, occf, m_count)
    h2 = h2 + hsc
    cl = jnp.clip(jnp.floor(pos2[:, 0] / 0.25), 0, 3).astype(jnp.int32) + 4 * jnp.clip(jnp.floor(pos2[:, 1] / 0.25), 0, 3).astype(jnp.int32)
    cl = jnp.where(occ, cl, 16)
    xf = jax.ops.segment_max(h2, cl, num_segments=17)[:16]
    xf = jnp.where(jnp.isfinite(xf), xf, 0.0)
    xf = xf.reshape(1, 16 * h2.shape[1])
    return _fc(xf, Wfc)


# speedup vs baseline: 2.0203x; 2.0203x over previous
import functools

import jax
import jax.numpy as jnp
from jax import lax
from jax.experimental import pallas as pl
from jax.experimental.pallas import tpu as pltpu
from jax.experimental.pallas import tpu_sc as plsc

S0 = 16.0 / 346.0
S1 = 12.0 / 260.0
NGRID = 22
NG2 = NGRID * NGRID
NG4 = NG2 * NG2
NP2 = 234368
NS = 16
CH = 128
PC = 512
GRID = 16

_SC_PARAMS = pltpu.CompilerParams(
    needs_layout_passes=False, use_tc_tiling_on_sc=False)


def _mesh():
    return plsc.VectorSubcoreMesh(
        core_axis_name="c", subcore_axis_name="s", num_cores=1)


def _chunk_range(wid, total_chunks):
    q, r = divmod(total_chunks, NS)
    start = wid * q + jnp.minimum(wid, r)
    cnt = q + jnp.where(wid < r, 1, 0)
    return start, start + cnt


def _fill_rows(ref, val16, nrows, col0=0):
    iota16 = lax.iota(jnp.int32, 16)

    def body(i, _):
        plsc.store_scatter(ref, [jnp.full((16,), i, jnp.int32), iota16 + col0],
                           val16)
        return 0
    lax.fori_loop(0, nrows, body, 0)


def _edge_pass(y, src, dst, u0, u1, cp, flag16):
    NPAD = cp.shape[0]
    E = src.shape[0]
    total_chunks = E // CH
    rows_per = NPAD // NS
    pres_per = NP2 // NS

    @functools.partial(
        pl.kernel,
        out_type=[
            jax.ShapeDtypeStruct((NPAD, 16), jnp.float32),
            jax.ShapeDtypeStruct((NP2,), jnp.float32),
        ],
        mesh=_mesh(),
        compiler_params=_SC_PARAMS,
        scratch_types=[
            pltpu.VMEM((NPAD,), jnp.int32),
            pltpu.VMEM((CH,), jnp.int32),
            pltpu.VMEM((CH,), jnp.int32),
            pltpu.VMEM((CH,), jnp.float32),
            pltpu.VMEM((CH,), jnp.float32),
            pltpu.VMEM((CH,), jnp.int32),
            pltpu.VMEM((CH,), jnp.float32),
            pltpu.VMEM((CH, 64), jnp.float32),
            pltpu.VMEM((CH, 16), jnp.float32),
            pltpu.VMEM((128, 16), jnp.float32),
            pltpu.VMEM((2048,), jnp.float32),
            pltpu.VMEM((16,), jnp.int32),
            pltpu.VMEM_SHARED((NPAD, 16), jnp.float32),
            pltpu.VMEM_SHARED((NP2,), jnp.float32),
            pltpu.SemaphoreType.DMA,
        ],
    )
    def k(y_hbm, src_hbm, dst_hbm, u0_hbm, u1_hbm, cp_hbm, flag_hbm,
          acc_out, pres_out,
          c_v, srcb, dstb, u0b, u1b, keysb, onesb, rowsb, msgb, zero2d,
          zero1d, flagb, acc_sh, pres_sh, sem):
        sid = lax.axis_index("s")
        wid = sid
        iota16 = lax.iota(jnp.int32, 16)
        zero16 = jnp.zeros((16,), jnp.float32)

        pltpu.sync_copy(flag_hbm, flagb)
        flag = flagb[pl.ds(0, 16)][0]

        _fill_rows(zero2d, zero16, 128)

        def z1(i, _):
            zero1d[pl.ds(i * 16, 16)] = jnp.zeros((16,), jnp.float32)
            return 0
        lax.fori_loop(0, 128, z1, 0)

        def ob(i, _):
            onesb[pl.ds(i * 16, 16)] = jnp.ones((16,), jnp.float32)
            return 0
        lax.fori_loop(0, CH // 16, ob, 0)

        nfull_a, rem_a = divmod(rows_per, 128)
        for i in range(nfull_a):
            pltpu.sync_copy(zero2d.at[pl.ds(0, 128)],
                            acc_sh.at[pl.ds(sid * rows_per + i * 128, 128)])
        if rem_a:
            pltpu.sync_copy(zero2d.at[pl.ds(0, rem_a)],
                            acc_sh.at[pl.ds(sid * rows_per + nfull_a * 128, rem_a)])
        nfull_p, rem_p = divmod(pres_per, 2048)
        for i in range(nfull_p):
            pltpu.sync_copy(zero1d,
                            pres_sh.at[pl.ds(sid * pres_per + i * 2048, 2048)])
        if rem_p:
            pltpu.sync_copy(zero1d.at[pl.ds(0, rem_p)],
                            pres_sh.at[pl.ds(sid * pres_per + nfull_p * 2048, rem_p)])

        @pl.when(flag == 0)
        def _():
            pltpu.sync_copy(cp_hbm, c_v)

        plsc.subcore_barrier()

        lo, hi = _chunk_range(wid, total_chunks)

        def chunk_body(ci, _):
            off = ci * CH
            pltpu.sync_copy(src_hbm.at[pl.ds(off, CH)], srcb)
            pltpu.sync_copy(dst_hbm.at[pl.ds(off, CH)], dstb)
            pltpu.sync_copy(u0_hbm.at[pl.ds(off, CH)], u0b)
            pltpu.sync_copy(u1_hbm.at[pl.ds(off, CH)], u1b)
            pltpu.async_copy(y_hbm.at[srcb], rowsb, sem).wait()
            for j in range(0, CH, 16):
                eids = j + iota16
                u0v = u0b[pl.ds(j, 16)]
                u1v = u1b[pl.ds(j, 16)]
                for o in range(16):
                    co = jnp.full((16,), o, jnp.int32)
                    r0 = plsc.load_gather(rowsb, [eids, co])
                    r1 = plsc.load_gather(rowsb, [eids, co + 16])
                    r2 = plsc.load_gather(rowsb, [eids, co + 32])
                    r3 = plsc.load_gather(rowsb, [eids, co + 48])
                    a = r0 + u0v * r1
                    b = r2 + u0v * r3
                    m = a + u1v * (b - a)
                    plsc.store_scatter(msgb, [eids, co], m)
            pltpu.sync_copy(msgb, acc_sh.at[dstb], add=True)

            @pl.when(flag == 0)
            def _():
                for j in range(0, CH, 16):
                    sidx = srcb[pl.ds(j, 16)]
                    didx = dstb[pl.ds(j, 16)]
                    c0 = plsc.load_gather(c_v, [sidx])
                    c1 = plsc.load_gather(c_v, [didx])
                    keys = c0 * NG2 + c1
                    keys = jnp.where(c0 == c1, NG4 + iota16, keys)
                    keysb[pl.ds(j, 16)] = keys
                pltpu.sync_copy(onesb, pres_sh.at[keysb], add=True)
            return 0

        lax.fori_loop(lo, hi, chunk_body, 0)
        plsc.subcore_barrier()

        pltpu.sync_copy(acc_sh.at[pl.ds(sid * rows_per, rows_per)],
                        acc_out.at[pl.ds(sid * rows_per, rows_per)])
        pltpu.sync_copy(pres_sh.at[pl.ds(sid * pres_per, pres_per)],
                        pres_out.at[pl.ds(sid * pres_per, pres_per)])

    return k(y, src, dst, u0, u1, cp, flag16)


def _pool_pass(h6, cp, pxp, pyp):
    NPAD = h6.shape[0]
    total_chunks = NPAD // CH

    @functools.partial(
        pl.kernel,
        out_type=[
            jax.ShapeDtypeStruct((NS, PC, 32), jnp.float32),
            jax.ShapeDtypeStruct((NS, PC, 16), jnp.float32),
        ],
        mesh=_mesh(),
        compiler_params=_SC_PARAMS,
        scratch_types=[
            pltpu.VMEM((CH, 32), jnp.float32),
            pltpu.VMEM((CH,), jnp.int32),
            pltpu.VMEM((CH,), jnp.float32),
            pltpu.VMEM((CH,), jnp.float32),
            pltpu.VMEM((PC, 32), jnp.float32),
            pltpu.VMEM((PC, 16), jnp.float32),
        ],
    )
    def k(h6_hbm, c_hbm, px_hbm, py_hbm, max_out, sum_out,
          h6b, cb, pxb, pyb, maxt, sumt):
        sid = lax.axis_index("s")
        wid = sid
        iota16 = lax.iota(jnp.int32, 16)
        ninf = jnp.full((16,), -jnp.inf, jnp.float32)
        zero16 = jnp.zeros((16,), jnp.float32)
        oh0 = jnp.where(iota16 == 0, 1.0, 0.0).astype(jnp.float32)
        oh1 = jnp.where(iota16 == 1, 1.0, 0.0).astype(jnp.float32)
        oh2 = jnp.where(iota16 == 2, 1.0, 0.0).astype(jnp.float32)

        _fill_rows(maxt, ninf, PC, col0=0)
        _fill_rows(maxt, ninf, PC, col0=16)
        _fill_rows(sumt, zero16, PC)

        lo, hi = _chunk_range(wid, total_chunks)

        def chunk_body(ci, _):
            off = ci * CH
            pltpu.sync_copy(h6_hbm.at[pl.ds(off, CH)], h6b)
            pltpu.sync_copy(c_hbm.at[pl.ds(off, CH)], cb)
            pltpu.sync_copy(px_hbm.at[pl.ds(off, CH)], pxb)
            pltpu.sync_copy(py_hbm.at[pl.ds(off, CH)], pyb)
            for j in range(0, CH, 16):
                c16 = cb[pl.ds(j, 16)]
                px16 = pxb[pl.ds(j, 16)]
                py16 = pyb[pl.ds(j, 16)]
                for t in range(16):
                    e = j + t
                    fe = jnp.full((16,), e, jnp.int32)
                    crow = jnp.full((16,), c16[t], jnp.int32)
                    h0 = plsc.load_gather(h6b, [fe, iota16])
                    h1 = plsc.load_gather(h6b, [fe, iota16 + 16])
                    m0 = plsc.load_gather(maxt, [crow, iota16])
                    m1 = plsc.load_gather(maxt, [crow, iota16 + 16])
                    plsc.store_scatter(maxt, [crow, iota16], jnp.maximum(m0, h0))
                    plsc.store_scatter(maxt, [crow, iota16 + 16],
                                       jnp.maximum(m1, h1))
                    srow = plsc.load_gather(sumt, [crow, iota16])
                    contrib = (jnp.full((16,), px16[t], jnp.float32) * oh0
                               + jnp.full((16,), py16[t], jnp.float32) * oh1
                               + oh2)
                    plsc.store_scatter(sumt, [crow, iota16], srow + contrib)
            return 0

        lax.fori_loop(lo, hi, chunk_body, 0)

        pltpu.sync_copy(maxt, max_out.at[sid])
        pltpu.sync_copy(sumt, sum_out.at[sid])

    return k(h6, cp, pxp, pyp)


def _elu(x):
    return jnp.where(x > 0, x, jnp.exp(jnp.minimum(x, 0.0)) - 1.0)


def _prep_kernel(px2_ref, py2_ref, n_ref, c2_ref):
    n = n_ref[0]
    rows, cols = px2_ref.shape
    flat = (lax.broadcasted_iota(jnp.int32, (rows, cols), 0) * cols
            + lax.broadcasted_iota(jnp.int32, (rows, cols), 1))
    c = (jnp.floor(px2_ref[...] / S0).astype(jnp.int32)
         + NGRID * jnp.floor(py2_ref[...] / S1).astype(jnp.int32))
    c2_ref[...] = jnp.where(flat < n, c, 490)


def _prep_c(px2, py2, n):
    rows, cols = px2.shape
    return pl.pallas_call(
        _prep_kernel,
        out_shape=jax.ShapeDtypeStruct((rows, cols), jnp.int32),
        in_specs=[pl.BlockSpec(memory_space=pltpu.VMEM),
                  pl.BlockSpec(memory_space=pltpu.VMEM),
                  pl.BlockSpec(memory_space=pltpu.SMEM)],
        out_specs=pl.BlockSpec(memory_space=pltpu.VMEM),
    )(px2, py2, jnp.array([n], jnp.int32))


def _y1_kernel(x2_ref, v_ref, e_ref, y1_ref):
    y1_ref[...] = x2_ref[...] * v_ref[...] + e_ref[...]


def _y1_table(xp, v, e):
    NPAD = xp.shape[0]
    br = NPAD // GRID
    return pl.pallas_call(
        _y1_kernel,
        grid=(GRID,),
        in_specs=[pl.BlockSpec((br, 1), lambda i: (i, 0)),
                  pl.BlockSpec((1, 64), lambda i: (0, 0)),
                  pl.BlockSpec((1, 64), lambda i: (0, 0))],
        out_specs=pl.BlockSpec((br, 64), lambda i: (i, 0)),
        out_shape=jax.ShapeDtypeStruct((NPAD, 64), jnp.float32),
    )(xp.reshape(NPAD, 1), v.reshape(1, 64), e.reshape(1, 64))


def _rdeg_kernel(a0_ref, rdeg_ref):
    deg = a0_ref[...][:, 8:9]
    rdeg_ref[...] = jnp.broadcast_to(1.0 / jnp.clip(deg, 1.0, None),
                                     rdeg_ref.shape)


def _hstats_kernel(ch, a0_ref, rdeg_ref, h_ref, st_ref):
    i = pl.program_id(0)
    s = a0_ref[...][:, :ch]
    rdeg = rdeg_ref[...][:, 0:1]
    h = _elu(s * rdeg)
    h_ref[...] = h

    @pl.when(i == 0)
    def _():
        st_ref[...] = jnp.zeros_like(st_ref)
    st = jnp.stack([jnp.sum(h, axis=0), jnp.sum(h * h, axis=0)], axis=0)
    st_ref[...] = st_ref[...] + st


def _hstats(a0, rdeg, ch):
    NPAD = a0.shape[0]
    br = NPAD // GRID
    cin = a0.shape[1]
    return pl.pallas_call(
        functools.partial(_hstats_kernel, ch),
        grid=(GRID,),
        in_specs=[
            pl.BlockSpec((br, cin), lambda i: (i, 0)),
            pl.BlockSpec((br, 8), lambda i: (i, 0)),
        ],
        out_specs=[
            pl.BlockSpec((br, ch), lambda i: (i, 0)),
            pl.BlockSpec((2, ch), lambda i: (0, 0)),
        ],
        out_shape=[
            jax.ShapeDtypeStruct((NPAD, ch), jnp.float32),
            jax.ShapeDtypeStruct((2, ch), jnp.float32),
        ],
    )(a0, rdeg)


def _bnmm_kernel(has_res, want_y, want_h, h_ref, st_ref, g_ref, b_ref,
                 w_ref, n_ref, *rest):
    idx = 0
    hsc_ref = None
    if has_res:
        hsc_ref = rest[idx]
        idx += 1
    y_ref = None
    if want_y:
        y_ref = rest[idx]
        idx += 1
    h_out_ref = rest[idx] if want_h else None

    nf = n_ref[0]
    st = st_ref[...]
    m = st[0:1, :] / nf
    v = st[1:2, :] / nf - m * m
    h = g_ref[...][None, :] * (h_ref[...] - m) / jnp.sqrt(v + 1e-5) \
        + b_ref[...][None, :]
    if has_res:
        h = h + hsc_ref[...]
    if want_y:
        y_ref[...] = h @ w_ref[...]
    if want_h:
        h_out_ref[...] = h


def _bnmm(h, st, g, b, w, n, hsc=None, want_y=True, want_h=False):
    NPAD = h.shape[0]
    ch = h.shape[1]
    br = NPAD // GRID
    if w is None:
        w = jnp.zeros((ch, 8), jnp.float32)
    ycols = w.shape[1]
    in_specs = [
        pl.BlockSpec((br, ch), lambda i: (i, 0)),
        pl.BlockSpec((2, ch), lambda i: (0, 0)),
        pl.BlockSpec(memory_space=pltpu.VMEM),
        pl.BlockSpec(memory_space=pltpu.VMEM),
        pl.BlockSpec(memory_space=pltpu.VMEM),
        pl.BlockSpec(memory_space=pltpu.SMEM),
    ]
    args = [h, st, g, b, w, jnp.array([float(n)], jnp.float32)]
    if hsc is not None:
        in_specs.append(pl.BlockSpec((br, ch), lambda i: (i, 0)))
        args.append(hsc)
    out_specs, out_shape = [], []
    if want_y:
        out_specs.append(pl.BlockSpec((br, ycols), lambda i: (i, 0)))
        out_shape.append(jax.ShapeDtypeStruct((NPAD, ycols), jnp.float32))
    if want_h:
        out_specs.append(pl.BlockSpec((br, ch), lambda i: (i, 0)))
        out_shape.append(jax.ShapeDtypeStruct((NPAD, ch), jnp.float32))
    return pl.pallas_call(
        functools.partial(_bnmm_kernel, hsc is not None, want_y, want_h),
        grid=(GRID,),
        in_specs=in_specs,
        out_specs=out_specs,
        out_shape=out_shape,
    )(*args)


def _coarse_kernel(maxr_ref, sums_ref, praw_ref, w6_ref, w7_ref,
                   g6_ref, b6_ref, g7_ref, b7_ref, wfc_ref, out_ref):
    sums = jnp.sum(sums_ref[...], axis=0)
    cnt = sums[:, 2]
    rows = lax.broadcasted_iota(jnp.int32, cnt.shape, 0)
    occ = (cnt > 0) & (rows < NG2)
    occf = occ.astype(jnp.float32)
    m_count = jnp.sum(occf)
    h2 = jnp.max(maxr_ref[...], axis=0)
    h2 = jnp.where(occ[:, None], h2, 0.0)
    ccl = jnp.clip(cnt, 1.0, None)
    p2x = jnp.where(occ, sums[:, 0] / ccl, 0.0)
    p2y = jnp.where(occ, sums[:, 1] / ccl, 0.0)
    presf = (praw_ref[...] > 0).astype(jnp.float32)
    dx = p2x[:, None] - p2x[None, :]
    dy = p2y[:, None] - p2y[None, :]
    mx = jnp.maximum(jnp.max(jnp.abs(dx) * presf), jnp.max(jnp.abs(dy) * presf))
    norm = 2.0 * mx
    u0 = dx / norm + 0.5
    u1 = dy / norm + 0.5
    b0 = (1.0 - u0) * (1.0 - u1)
    b1 = u0 * (1.0 - u1)
    b2 = (1.0 - u0) * u1
    b3 = u0 * u1
    deg2 = jnp.sum(presf, axis=0)
    rdeg2 = 1.0 / jnp.clip(deg2, 1.0, None)

    def conv(h, w_ref, g_ref, b_ref):
        w = w_ref[...]
        agg = ((presf * b0).T @ (h @ w[0])
               + (presf * b1).T @ (h @ w[1])
               + (presf * b2).T @ (h @ w[2])
               + (presf * b3).T @ (h @ w[3]))
        agg = agg * rdeg2[:, None]
        hh = _elu(agg)
        m = jnp.sum(hh * occf[:, None], axis=0) / m_count
        v = jnp.sum(((hh - m[None, :]) ** 2) * occf[:, None], axis=0) / m_count
        return g_ref[...][None, :] * (hh - m[None, :]) / jnp.sqrt(v + 1e-5) \
            + b_ref[...][None, :]

    hsc = h2
    h2 = conv(h2, w6_ref, g6_ref, b6_ref)
    h2 = conv(h2, w7_ref, g7_ref, b7_ref)
    h2 = h2 + hsc
    cl = (jnp.clip(jnp.floor(p2x / 0.25), 0, 3).astype(jnp.int32)
          + 4 * jnp.clip(jnp.floor(p2y / 0.25), 0, 3).astype(jnp.int32))
    cl = jnp.where(occ, cl, 16)
    wfc = wfc_ref[...]
    out = jnp.zeros((1, 10), jnp.float32)
    for jj in range(16):
        mj = jnp.max(jnp.where((cl == jj)[:, None], h2, -jnp.inf),
                     axis=0, keepdims=True)
        mj = jnp.where(jnp.isfinite(mj), mj, 0.0)
        out = out + mj @ wfc[jj * 32:(jj + 1) * 32]
    out_ref[...] = out


def _coarse(maxr, sums, praw, W6, W7, g6, b6, g7, b7, Wfc):
    return pl.pallas_call(
        _coarse_kernel,
        out_shape=jax.ShapeDtypeStruct((1, 10), jnp.float32),
    )(maxr, sums, praw, W6, W7, g6, b6, g7, b7, Wfc)


def _wcat(W, lo=0, hi=None):
    if hi is None:
        hi = W.shape[2]
    return jnp.concatenate(
        [W[0][:, lo:hi], (W[1] - W[0])[:, lo:hi],
         W[2][:, lo:hi], (W[3] - W[2])[:, lo:hi]], axis=1)


def kernel(x, edge_attr, pos, W1, W2, W2_1, W3, W4, W5, W6, W7, g1, b1, g2, b2, g2_1, b2_1, g3, b3, g4, b4, g5, b5, g6, b6, g7, b7, Wfc, edge_index, batch):
    N = x.shape[0]
    NPAD = ((N + 127) // 128) * 128

    esrc = edge_index[0]
    edst = edge_index[1]
    u0 = edge_attr[:, 0]
    u1 = edge_attr[:, 1]
    pxp = jnp.zeros((NPAD,), jnp.float32).at[:N].set(pos[:, 0])
    pyp = jnp.zeros((NPAD,), jnp.float32).at[:N].set(pos[:, 1])
    xp = jnp.zeros((NPAD,), jnp.float32).at[:N].set(x[:, 0])

    c2 = _prep_c(pxp.reshape(NPAD // 128, 128), pyp.reshape(NPAD // 128, 128), N)
    cp = c2.reshape(NPAD)

    w1 = W1.reshape(4, 8)
    zpad = jnp.zeros((8,), jnp.float32)
    v = jnp.concatenate([w1[0], zpad, w1[1] - w1[0], zpad,
                         w1[2], zpad, w1[3] - w1[2], zpad])
    e = jnp.zeros((64,), jnp.float32).at[8].set(1.0).at[40].set(1.0)
    y1 = _y1_table(xp, v, e)

    w2c = _wcat(W2)
    w21c = _wcat(W2_1)
    w3c = _wcat(W3)
    w4c = _wcat(W4)
    w5c = jnp.concatenate([_wcat(W5, 0, 16), _wcat(W5, 16, 32)], axis=1)

    zeros64 = jnp.zeros((NPAD, 64), jnp.float32)
    zeros16 = jnp.zeros((NPAD, 16), jnp.float32)
    zeros8 = jnp.zeros((NPAD, 8), jnp.float32)

    def b_l1(acc, y_pend, h3, acc6a, rdeg):
        rdeg_n = pl.pallas_call(
            _rdeg_kernel,
            grid=(GRID,),
            in_specs=[pl.BlockSpec((NPAD // GRID, 16), lambda i: (i, 0))],
            out_specs=pl.BlockSpec((NPAD // GRID, 8), lambda i: (i, 0)),
            out_shape=jax.ShapeDtypeStruct((NPAD, 8), jnp.float32),
        )(acc)
        h, st = _hstats(acc, rdeg_n, 8)
        (y2,) = _bnmm(h, st, g1, b1, w2c, N)
        return y2, y_pend, h3, acc6a, rdeg_n

    def b_mid(gv, bv, wv):
        def f(acc, y_pend, h3, acc6a, rdeg):
            h, st = _hstats(acc, rdeg, 16)
            (yn,) = _bnmm(h, st, gv, bv, wv, N)
            return yn, y_pend, h3, acc6a, rdeg
        return f

    def b_l3(acc, y_pend, h3, acc6a, rdeg):
        h, st = _hstats(acc, rdeg, 16)
        yn, h3n = _bnmm(h, st, g2_1, b2_1, w3c, N, want_h=True)
        return yn, y_pend, h3n, acc6a, rdeg

    def b_l5(acc, y_pend, h3, acc6a, rdeg):
        h, st = _hstats(acc, rdeg, 16)
        (y6,) = _bnmm(h, st, g4, b4, w5c, N, hsc=h3)
        return y6[:, :64], y6[:, 64:], h3, acc6a, rdeg

    def b_l6a(acc, y_pend, h3, acc6a, rdeg):
        return y_pend, y_pend, h3, acc, rdeg

    def b_l6b(acc, y_pend, h3, acc6a, rdeg):
        return y_pend, y_pend, h3, acc6a, rdeg

    branches = [b_l1, b_mid(g2, b2, w21c), b_l3, b_mid(g3, b3, w4c),
                b_l5, b_l6a, b_l6b]

    def scan_body(carry, step):
        y_cur, y_pend, h3, acc6a, rdeg, _ = carry
        flag16 = jnp.full((16,), step, jnp.int32)
        acc, pres = _edge_pass(y_cur, esrc, edst, u0, u1, cp, flag16)
        y_n, y_p, h3n, a6a, rd = lax.switch(step, branches,
                                            acc, y_pend, h3, acc6a, rdeg)
        return (y_n, y_p, h3n, a6a, rd, acc), pres

    carry0 = (y1, zeros64, zeros16, zeros16, zeros8, zeros16)
    (y_f, y_pf, h3f, acc6a, rdeg, acc6b), pres_steps = lax.scan(
        scan_body, carry0, jnp.arange(7, dtype=jnp.int32))

    acc6 = jnp.concatenate([acc6a, acc6b], axis=1)
    h, st = _hstats(acc6, rdeg, 32)
    (h6,) = _bnmm(h, st, g5, b5, None, N, want_y=False, want_h=True)

    maxr, sums = _pool_pass(h6, cp, pxp, pyp)

    praw = pres_steps[0][:NG4].reshape(NG2, NG2)
    praw = jnp.pad(praw, ((0, PC - NG2), (0, PC - NG2)))

    return _coarse(maxr, sums, praw, W6, W7, g6, b6, g7, b7, Wfc)
